# baseline (speedup 1.0000x reference)
import functools

import jax
import jax.numpy as jnp
from jax import lax
from jax.experimental import pallas as pl
from jax.experimental.pallas import tpu as pltpu
from jax.experimental.pallas import tpu_sc as plsc

_N = 10000
_E = 320000
_D = 128
_EPS = 1e-5

_NC = 2
_NS = 16
_NW = _NC * _NS
_EPW = _E // _NW
_BATCH = 128
_EPWP = 10240
_NPASS = 2
_NBP = _EPWP // _BATCH // _NPASS
_NP = 10112
_RSUB = _NP // _NS


def _sc_mesh():
    return plsc.VectorSubcoreMesh(
        core_axis_name="c", subcore_axis_name="s",
        num_cores=_NC, num_subcores=_NS)


@functools.lru_cache(maxsize=None)
def _make_sc_agg():
    scratch = [
        pltpu.VMEM_SHARED((_NP, _D), jnp.float32),
        pltpu.VMEM((_NBP, 1, _BATCH), jnp.int32),
        pltpu.VMEM((_NBP, 1, _BATCH), jnp.int32),
        pltpu.VMEM((2, _BATCH, _D), jnp.float32),
        pltpu.SemaphoreType.DMA,
        pltpu.SemaphoreType.DMA,
        pltpu.SemaphoreType.DMA,
        pltpu.SemaphoreType.DMA,
    ]

    def body(src_hbm, dst_hbm, x_hbm, zacc_hbm, acc_out,
             acc_sh, src_v, dst_v, stage, gsem0, gsem1, ssem0, ssem1):
        c = lax.axis_index("c")
        s = lax.axis_index("s")
        w = s * _NC + c
        gsems = (gsem0, gsem1)
        ssems = (ssem0, ssem1)

        pltpu.sync_copy(zacc_hbm.at[pl.ds(s * _RSUB, _RSUB)],
                        acc_sh.at[pl.ds(s * _RSUB, _RSUB)])
        plsc.subcore_barrier()

        for p in range(_NPASS):
            pltpu.sync_copy(src_hbm.at[w, p], src_v)
            pltpu.sync_copy(dst_hbm.at[w, p], dst_v)
            pltpu.async_copy(x_hbm.at[src_v.at[0, 0]], stage.at[0], gsem0)
            pltpu.async_copy(x_hbm.at[src_v.at[1, 0]], stage.at[1], gsem1)

            @pl.loop(0, _NBP, step=2)
            def _(j2):
                for b in range(2):
                    j = j2 + b
                    pltpu.make_async_copy(
                        x_hbm.at[src_v.at[j, 0]], stage.at[b],
                        gsems[b]).wait()
                    pltpu.async_copy(
                        stage.at[b], acc_sh.at[dst_v.at[j, 0]], ssems[b],
                        add=True)
                    pltpu.make_async_copy(
                        stage.at[b], acc_sh.at[dst_v.at[j, 0]],
                        ssems[b]).wait()

                    @pl.when(j + 2 < _NBP)
                    def _():
                        pltpu.async_copy(
                            x_hbm.at[src_v.at[j + 2, 0]], stage.at[b],
                            gsems[b])

        plsc.subcore_barrier()
        pltpu.sync_copy(acc_sh.at[pl.ds(s * _RSUB, _RSUB)],
                        acc_out.at[c, pl.ds(s * _RSUB, _RSUB)])

    return pl.kernel(
        body,
        out_type=jax.ShapeDtypeStruct((_NC, _NP, _D), jnp.float32),
        mesh=_sc_mesh(), scratch_types=scratch)


@functools.lru_cache(maxsize=None)
def _make_sc_deg():
    scratch = [
        pltpu.VMEM_SHARED((_NP, _D), jnp.float32),
        pltpu.VMEM((_NBP, 1, _BATCH), jnp.int32),
        pltpu.VMEM((_BATCH, _D), jnp.float32),
        pltpu.SemaphoreType.DMA,
        pltpu.SemaphoreType.DMA,
    ]

    def body(dst_hbm, zacc_hbm, ones_hbm, deg_out,
             deg_sh, dst_v, ones_v, dsem0, dsem1):
        c = lax.axis_index("c")
        s = lax.axis_index("s")
        w = s * _NC + c
        dsems = (dsem0, dsem1)

        pltpu.sync_copy(ones_hbm, ones_v)
        pltpu.sync_copy(zacc_hbm.at[pl.ds(s * _RSUB, _RSUB)],
                        deg_sh.at[pl.ds(s * _RSUB, _RSUB)])
        plsc.subcore_barrier()

        for p in range(_NPASS):
            pltpu.sync_copy(dst_hbm.at[w, p], dst_v)

            @pl.loop(0, _NBP, step=2)
            def _(j2):
                for b in range(2):
                    pltpu.async_copy(
                        ones_v, deg_sh.at[dst_v.at[j2 + b, 0]], dsems[b],
                        add=True)
                for b in range(2):
                    pltpu.make_async_copy(
                        ones_v, deg_sh.at[dst_v.at[j2 + b, 0]],
                        dsems[b]).wait()

        plsc.subcore_barrier()
        pltpu.sync_copy(deg_sh.at[pl.ds(s * _RSUB, _RSUB)],
                        deg_out.at[c, pl.ds(s * _RSUB, _RSUB)])

    return pl.kernel(
        body,
        out_type=jax.ShapeDtypeStruct((_NC, _NP, _D), jnp.float32),
        mesh=_sc_mesh(), scratch_types=scratch)


def _tc_layer(x, accp, degp, Wl, bl, Wr, g, b, rm, rv):
    R = 1000
    grid = (_N // R,)

    def body(x_ref, acc_ref, deg_ref, wl_ref, bl_ref, wr_ref,
             g_ref, b_ref, rm_ref, rv_ref, o_ref):
        xb = x_ref[...]
        agg = acc_ref[0] + acc_ref[1]
        deg = deg_ref[0, :, 0:1] + deg_ref[1, :, 0:1]
        mean = agg / jnp.maximum(deg, 1.0)
        h = lax.dot_general(mean, wl_ref[...], (((1,), (1,)), ((), ())),
                            preferred_element_type=jnp.float32)
        h = h + lax.dot_general(xb, wr_ref[...], (((1,), (1,)), ((), ())),
                                preferred_element_type=jnp.float32)
        h = h + bl_ref[...]
        scale = g_ref[...] * lax.rsqrt(rv_ref[...] + _EPS)
        shift = b_ref[...] - rm_ref[...] * scale
        h = h * scale + shift
        o_ref[...] = xb + jnp.maximum(h, 0.0)

    full = lambda i: (0, 0)
    part3 = lambda i: (0, i, 0)
    return pl.pallas_call(
        body,
        grid=grid,
        in_specs=[
            pl.BlockSpec((R, _D), lambda i: (i, 0)),
            pl.BlockSpec((_NC, R, _D), part3),
            pl.BlockSpec((_NC, R, _D), part3),
            pl.BlockSpec((_D, _D), full),
            pl.BlockSpec((1, _D), full),
            pl.BlockSpec((_D, _D), full),
            pl.BlockSpec((1, _D), full),
            pl.BlockSpec((1, _D), full),
            pl.BlockSpec((1, _D), full),
            pl.BlockSpec((1, _D), full),
        ],
        out_specs=pl.BlockSpec((R, _D), lambda i: (i, 0)),
        out_shape=jax.ShapeDtypeStruct((_N, _D), jnp.float32),
    )(x, accp, degp, Wl, bl, Wr, g, b, rm, rv)


def kernel(edge_index, emb, Wl0, bl0, Wr0, g0, b0, rm0, rv0,
           Wl1, bl1, Wr1, g1, b1, rm1, rv1):
    pad = _EPWP - _EPW
    shape5 = (_NW, _NPASS, _NBP, 1, _BATCH)
    src = jnp.pad(edge_index[0].reshape(_NW, _EPW), ((0, 0), (0, pad)),
                  constant_values=0).reshape(shape5)
    spread = _N + (jnp.arange(pad, dtype=jnp.int32) % (_NP - _N))
    dst = jnp.concatenate(
        [edge_index[1].reshape(_NW, _EPW),
         jnp.broadcast_to(spread, (_NW, pad))], axis=1).reshape(shape5)
    zacc = jnp.zeros((_NP, _D), jnp.float32)
    ones = jnp.ones((_BATCH, _D), jnp.float32)
    r = lambda v: v.reshape(1, _D)

    degp = _make_sc_deg()(dst, zacc, ones)
    accp0 = _make_sc_agg()(src, dst, emb, zacc)
    x1 = _tc_layer(emb, accp0, degp, Wl0, r(bl0), Wr0,
                   r(g0), r(b0), r(rm0), r(rv0))
    accp1 = _make_sc_agg()(src, dst, x1, zacc)
    x2 = _tc_layer(x1, accp1, degp, Wl1, r(bl1), Wr1,
                   r(g1), r(b1), r(rm1), r(rv1))
    return x2


# speedup vs baseline: 2.2998x; 2.2998x over previous
import functools

import jax
import jax.numpy as jnp
from jax import lax
from jax.experimental import pallas as pl
from jax.experimental.pallas import tpu as pltpu
from jax.experimental.pallas import tpu_sc as plsc

_N = 10000
_E = 320000
_D = 128
_EPS = 1e-5

_NC = 2
_NS = 16
_NW = _NC * _NS
_EPW = _E // _NW
_BATCH = 100
_NPASS = 5
_NBP = _EPW // _BATCH // _NPASS
_NP = 10112
_RSUB = _NP // _NS


def _sc_mesh():
    return plsc.VectorSubcoreMesh(
        core_axis_name="c", subcore_axis_name="s",
        num_cores=_NC, num_subcores=_NS)


@functools.lru_cache(maxsize=None)
def _make_sc_agg():
    scratch = [
        pltpu.VMEM_SHARED((_NP, _D), jnp.float32),
        pltpu.VMEM((_NBP, 1, _BATCH), jnp.int32),
        pltpu.VMEM((_NBP, 1, _BATCH), jnp.int32),
        pltpu.VMEM((2, _BATCH, _D), jnp.float32),
        pltpu.SemaphoreType.DMA,
        pltpu.SemaphoreType.DMA,
        pltpu.SemaphoreType.DMA,
        pltpu.SemaphoreType.DMA,
    ]

    def body(src_hbm, dst_hbm, x_hbm, zacc_hbm, acc_out,
             acc_sh, src_v, dst_v, stage, gsem0, gsem1, ssem0, ssem1):
        c = lax.axis_index("c")
        s = lax.axis_index("s")
        w = s * _NC + c
        gsems = (gsem0, gsem1)
        ssems = (ssem0, ssem1)

        pltpu.sync_copy(zacc_hbm.at[pl.ds(s * _RSUB, _RSUB)],
                        acc_sh.at[pl.ds(s * _RSUB, _RSUB)])
        plsc.subcore_barrier()

        for p in range(_NPASS):
            pltpu.sync_copy(src_hbm.at[w, p], src_v)
            pltpu.sync_copy(dst_hbm.at[w, p], dst_v)
            pltpu.async_copy(x_hbm.at[src_v.at[0, 0]], stage.at[0], gsem0)
            pltpu.async_copy(x_hbm.at[src_v.at[1, 0]], stage.at[1], gsem1)

            @pl.loop(0, _NBP, step=2)
            def _(j2):
                for b in range(2):
                    j = j2 + b
                    pltpu.make_async_copy(
                        x_hbm.at[src_v.at[j, 0]], stage.at[b],
                        gsems[b]).wait()
                    pltpu.async_copy(
                        stage.at[b], acc_sh.at[dst_v.at[j, 0]], ssems[b],
                        add=True)
                    pltpu.make_async_copy(
                        stage.at[b], acc_sh.at[dst_v.at[j, 0]],
                        ssems[b]).wait()
                    jn = jnp.minimum(j + 2, _NBP - 1)
                    pltpu.async_copy(
                        x_hbm.at[src_v.at[jn, 0]], stage.at[b], gsems[b])

            for b in range(2):
                pltpu.make_async_copy(
                    x_hbm.at[src_v.at[_NBP - 1, 0]], stage.at[b],
                    gsems[b]).wait()

        plsc.subcore_barrier()
        pltpu.sync_copy(acc_sh.at[pl.ds(s * _RSUB, _RSUB)],
                        acc_out.at[c, pl.ds(s * _RSUB, _RSUB)])

    return pl.kernel(
        body,
        out_type=jax.ShapeDtypeStruct((_NC, _NP, _D), jnp.float32),
        mesh=_sc_mesh(), scratch_types=scratch)


@functools.lru_cache(maxsize=None)
def _make_sc_deg():
    scratch = [
        pltpu.VMEM_SHARED((_NP, _D), jnp.float32),
        pltpu.VMEM((_NBP, 1, _BATCH), jnp.int32),
        pltpu.VMEM((_BATCH, _D), jnp.float32),
        pltpu.SemaphoreType.DMA,
        pltpu.SemaphoreType.DMA,
    ]

    def body(dst_hbm, zacc_hbm, ones_hbm, deg_out,
             deg_sh, dst_v, ones_v, dsem0, dsem1):
        c = lax.axis_index("c")
        s = lax.axis_index("s")
        w = s * _NC + c
        dsems = (dsem0, dsem1)

        pltpu.sync_copy(ones_hbm, ones_v)
        pltpu.sync_copy(zacc_hbm.at[pl.ds(s * _RSUB, _RSUB)],
                        deg_sh.at[pl.ds(s * _RSUB, _RSUB)])
        plsc.subcore_barrier()

        for p in range(_NPASS):
            pltpu.sync_copy(dst_hbm.at[w, p], dst_v)

            @pl.loop(0, _NBP, step=2)
            def _(j2):
                for b in range(2):
                    pltpu.async_copy(
                        ones_v, deg_sh.at[dst_v.at[j2 + b, 0]], dsems[b],
                        add=True)
                for b in range(2):
                    pltpu.make_async_copy(
                        ones_v, deg_sh.at[dst_v.at[j2 + b, 0]],
                        dsems[b]).wait()

        plsc.subcore_barrier()
        pltpu.sync_copy(deg_sh.at[pl.ds(s * _RSUB, _RSUB)],
                        deg_out.at[c, pl.ds(s * _RSUB, _RSUB)])

    return pl.kernel(
        body,
        out_type=jax.ShapeDtypeStruct((_NC, _NP, _D), jnp.float32),
        mesh=_sc_mesh(), scratch_types=scratch)


def _tc_layer(x, accp, degp, Wl, bl, Wr, g, b, rm, rv):
    R = 1000
    grid = (_N // R,)

    def body(x_ref, acc_ref, deg_ref, wl_ref, bl_ref, wr_ref,
             g_ref, b_ref, rm_ref, rv_ref, o_ref):
        xb = x_ref[...]
        agg = acc_ref[0] + acc_ref[1]
        deg = deg_ref[0, :, 0:1] + deg_ref[1, :, 0:1]
        mean = agg / jnp.maximum(deg, 1.0)
        h = lax.dot_general(mean, wl_ref[...], (((1,), (1,)), ((), ())),
                            preferred_element_type=jnp.float32)
        h = h + lax.dot_general(xb, wr_ref[...], (((1,), (1,)), ((), ())),
                                preferred_element_type=jnp.float32)
        h = h + bl_ref[...]
        scale = g_ref[...] * lax.rsqrt(rv_ref[...] + _EPS)
        shift = b_ref[...] - rm_ref[...] * scale
        h = h * scale + shift
        o_ref[...] = xb + jnp.maximum(h, 0.0)

    full = lambda i: (0, 0)
    part3 = lambda i: (0, i, 0)
    return pl.pallas_call(
        body,
        grid=grid,
        in_specs=[
            pl.BlockSpec((R, _D), lambda i: (i, 0)),
            pl.BlockSpec((_NC, R, _D), part3),
            pl.BlockSpec((_NC, R, _D), part3),
            pl.BlockSpec((_D, _D), full),
            pl.BlockSpec((1, _D), full),
            pl.BlockSpec((_D, _D), full),
            pl.BlockSpec((1, _D), full),
            pl.BlockSpec((1, _D), full),
            pl.BlockSpec((1, _D), full),
            pl.BlockSpec((1, _D), full),
        ],
        out_specs=pl.BlockSpec((R, _D), lambda i: (i, 0)),
        out_shape=jax.ShapeDtypeStruct((_N, _D), jnp.float32),
    )(x, accp, degp, Wl, bl, Wr, g, b, rm, rv)


def kernel(edge_index, emb, Wl0, bl0, Wr0, g0, b0, rm0, rv0,
           Wl1, bl1, Wr1, g1, b1, rm1, rv1):
    shape5 = (_NW, _NPASS, _NBP, 1, _BATCH)
    src = edge_index[0].reshape(shape5)
    dst = edge_index[1].reshape(shape5)
    zacc = jnp.zeros((_NP, _D), jnp.float32)
    ones = jnp.ones((_BATCH, _D), jnp.float32)
    r = lambda v: v.reshape(1, _D)

    degp = _make_sc_deg()(dst, zacc, ones)
    accp0 = _make_sc_agg()(src, dst, emb, zacc)
    x1 = _tc_layer(emb, accp0, degp, Wl0, r(bl0), Wr0,
                   r(g0), r(b0), r(rm0), r(rv0))
    accp1 = _make_sc_agg()(src, dst, x1, zacc)
    x2 = _tc_layer(x1, accp1, degp, Wl1, r(bl1), Wr1,
                   r(g1), r(b1), r(rm1), r(rv1))
    return x2


# speedup vs baseline: 2.5116x; 1.0921x over previous
import functools

import jax
import jax.numpy as jnp
from jax import lax
from jax.experimental import pallas as pl
from jax.experimental.pallas import tpu as pltpu
from jax.experimental.pallas import tpu_sc as plsc

_N = 10000
_E = 320000
_D = 128
_EPS = 1e-5

_NC = 2
_NS = 16
_NW = _NC * _NS
_EPW = _E // _NW
_BATCH = 100
_NPASS = 5
_NBP = _EPW // _BATCH // _NPASS
_NP = 10112
_RSUB = _NP // _NS


def _sc_mesh():
    return plsc.VectorSubcoreMesh(
        core_axis_name="c", subcore_axis_name="s",
        num_cores=_NC, num_subcores=_NS)


@functools.lru_cache(maxsize=None)
def _make_sc_agg():
    scratch = [
        pltpu.VMEM_SHARED((_NP, _D), jnp.float32),
        pltpu.VMEM((_NBP, 1, _BATCH), jnp.int32),
        pltpu.VMEM((_NBP, 1, _BATCH), jnp.int32),
        pltpu.VMEM((3, _BATCH, _D), jnp.float32),
        pltpu.SemaphoreType.DMA,
        pltpu.SemaphoreType.DMA,
        pltpu.SemaphoreType.DMA,
        pltpu.SemaphoreType.DMA,
        pltpu.SemaphoreType.DMA,
        pltpu.SemaphoreType.DMA,
    ]

    def body(src_hbm, dst_hbm, x_hbm, zacc_hbm, acc_out, acc_sh,
             src_v, dst_v, stage, gsem0, gsem1, gsem2,
             ssem0, ssem1, ssem2):
        c = lax.axis_index("c")
        s = lax.axis_index("s")
        w = s * _NC + c
        gsems = (gsem0, gsem1, gsem2)
        ssems = (ssem0, ssem1, ssem2)

        def wait_gather(j, b):
            pltpu.make_async_copy(
                x_hbm.at[src_v.at[j, 0]], stage.at[b], gsems[b]).wait()

        def issue_scatter(j, b):
            pltpu.async_copy(
                stage.at[b], acc_sh.at[dst_v.at[j, 0]], ssems[b],
                add=True)

        def wait_scatter(j, b):
            pltpu.make_async_copy(
                stage.at[b], acc_sh.at[dst_v.at[j, 0]], ssems[b]).wait()

        def issue_gather(j, b):
            pltpu.async_copy(
                x_hbm.at[src_v.at[j, 0]], stage.at[b], gsems[b])

        pltpu.sync_copy(zacc_hbm.at[pl.ds(s * _RSUB, _RSUB)],
                        acc_sh.at[pl.ds(s * _RSUB, _RSUB)])
        plsc.subcore_barrier()

        for p in range(_NPASS):
            pltpu.sync_copy(src_hbm.at[w, p], src_v)
            pltpu.sync_copy(dst_hbm.at[w, p], dst_v)
            issue_gather(0, 0)
            issue_gather(1, 1)

            @pl.loop(0, _NBP - 2, step=3)
            def _(j3):
                for u in range(3):
                    j = j3 + u
                    b = u
                    wait_gather(j, b)
                    issue_scatter(j, b)

                    @pl.when(j > 0)
                    def _():
                        wait_scatter(j - 1, (u + 2) % 3)

                    @pl.when(j + 2 < _NBP)
                    def _():
                        issue_gather(j + 2, (u + 2) % 3)

            wait_gather(_NBP - 2, 0)
            issue_scatter(_NBP - 2, 0)
            wait_scatter(_NBP - 3, 2)
            wait_gather(_NBP - 1, 1)
            issue_scatter(_NBP - 1, 1)
            wait_scatter(_NBP - 2, 0)
            wait_scatter(_NBP - 1, 1)

        plsc.subcore_barrier()
        pltpu.sync_copy(acc_sh.at[pl.ds(s * _RSUB, _RSUB)],
                        acc_out.at[c, pl.ds(s * _RSUB, _RSUB)])

    return pl.kernel(
        body,
        out_type=jax.ShapeDtypeStruct((_NC, _NP, _D), jnp.float32),
        mesh=_sc_mesh(), scratch_types=scratch)


@functools.lru_cache(maxsize=None)
def _make_sc_deg():
    scratch = [
        pltpu.VMEM_SHARED((_NP, _D), jnp.float32),
        pltpu.VMEM((_NBP, 1, _BATCH), jnp.int32),
        pltpu.VMEM((_BATCH, _D), jnp.float32),
        pltpu.SemaphoreType.DMA,
        pltpu.SemaphoreType.DMA,
    ]

    def body(dst_hbm, zacc_hbm, ones_hbm, deg_out,
             deg_sh, dst_v, ones_v, dsem0, dsem1):
        c = lax.axis_index("c")
        s = lax.axis_index("s")
        w = s * _NC + c
        dsems = (dsem0, dsem1)

        pltpu.sync_copy(ones_hbm, ones_v)
        pltpu.sync_copy(zacc_hbm.at[pl.ds(s * _RSUB, _RSUB)],
                        deg_sh.at[pl.ds(s * _RSUB, _RSUB)])
        plsc.subcore_barrier()

        for p in range(_NPASS):
            pltpu.sync_copy(dst_hbm.at[w, p], dst_v)

            @pl.loop(0, _NBP, step=2)
            def _(j2):
                for b in range(2):
                    pltpu.async_copy(
                        ones_v, deg_sh.at[dst_v.at[j2 + b, 0]], dsems[b],
                        add=True)
                for b in range(2):
                    pltpu.make_async_copy(
                        ones_v, deg_sh.at[dst_v.at[j2 + b, 0]],
                        dsems[b]).wait()

        plsc.subcore_barrier()
        pltpu.sync_copy(deg_sh.at[pl.ds(s * _RSUB, _RSUB)],
                        deg_out.at[c, pl.ds(s * _RSUB, _RSUB)])

    return pl.kernel(
        body,
        out_type=jax.ShapeDtypeStruct((_NC, _NP, _D), jnp.float32),
        mesh=_sc_mesh(), scratch_types=scratch)


def _tc_layer(x, accp, degp, Wl, bl, Wr, g, b, rm, rv):
    R = 1000
    grid = (_N // R,)

    def body(x_ref, acc_ref, deg_ref, wl_ref, bl_ref, wr_ref,
             g_ref, b_ref, rm_ref, rv_ref, o_ref):
        xb = x_ref[...]
        agg = acc_ref[0] + acc_ref[1]
        deg = deg_ref[0, :, 0:1] + deg_ref[1, :, 0:1]
        mean = agg / jnp.maximum(deg, 1.0)
        h = lax.dot_general(mean, wl_ref[...], (((1,), (1,)), ((), ())),
                            preferred_element_type=jnp.float32)
        h = h + lax.dot_general(xb, wr_ref[...], (((1,), (1,)), ((), ())),
                                preferred_element_type=jnp.float32)
        h = h + bl_ref[...]
        scale = g_ref[...] * lax.rsqrt(rv_ref[...] + _EPS)
        shift = b_ref[...] - rm_ref[...] * scale
        h = h * scale + shift
        o_ref[...] = xb + jnp.maximum(h, 0.0)

    full = lambda i: (0, 0)
    part3 = lambda i: (0, i, 0)
    return pl.pallas_call(
        body,
        grid=grid,
        in_specs=[
            pl.BlockSpec((R, _D), lambda i: (i, 0)),
            pl.BlockSpec((_NC, R, _D), part3),
            pl.BlockSpec((_NC, R, _D), part3),
            pl.BlockSpec((_D, _D), full),
            pl.BlockSpec((1, _D), full),
            pl.BlockSpec((_D, _D), full),
            pl.BlockSpec((1, _D), full),
            pl.BlockSpec((1, _D), full),
            pl.BlockSpec((1, _D), full),
            pl.BlockSpec((1, _D), full),
        ],
        out_specs=pl.BlockSpec((R, _D), lambda i: (i, 0)),
        out_shape=jax.ShapeDtypeStruct((_N, _D), jnp.float32),
    )(x, accp, degp, Wl, bl, Wr, g, b, rm, rv)


def kernel(edge_index, emb, Wl0, bl0, Wr0, g0, b0, rm0, rv0,
           Wl1, bl1, Wr1, g1, b1, rm1, rv1):
    shape5 = (_NW, _NPASS, _NBP, 1, _BATCH)
    src = edge_index[0].reshape(shape5)
    dst = edge_index[1].reshape(shape5)
    zacc = jnp.zeros((_NP, _D), jnp.float32)
    ones = jnp.ones((_BATCH, _D), jnp.float32)
    r = lambda v: v.reshape(1, _D)

    degp = _make_sc_deg()(dst, zacc, ones)
    accp0 = _make_sc_agg()(src, dst, emb, zacc)
    x1 = _tc_layer(emb, accp0, degp, Wl0, r(bl0), Wr0,
                   r(g0), r(b0), r(rm0), r(rv0))
    accp1 = _make_sc_agg()(src, dst, x1, zacc)
    x2 = _tc_layer(x1, accp1, degp, Wl1, r(bl1), Wr1,
                   r(g1), r(b1), r(rm1), r(rv1))
    return x2
